# baseline (speedup 1.0000x reference)
import functools

import jax
import jax.numpy as jnp
from jax import lax
from jax.experimental import pallas as pl
from jax.experimental.pallas import tpu as pltpu
from jax.experimental.pallas import tpu_sc as plsc

T = 2048
E = 64
H = 768
I = 1024
P = 2560
CHUNK = 64
BI = 512
W = 64
NC = 2
NSUB = 16
NW = NC * NSUB


def _router_body(l_ref, dest_ref, gate_ref, meta_ref):
    l = l_ref[...]
    m = jnp.max(l, axis=1, keepdims=True)
    s = jnp.sum(jnp.exp(l - m), axis=1, keepdims=True)
    gate_ref[...] = 1.0 / s

    oh = (l == m).astype(jnp.float32)
    tri_e = (lax.broadcasted_iota(jnp.int32, (E, E), 0)
             <= lax.broadcasted_iota(jnp.int32, (E, E), 1)).astype(jnp.float32)
    ecum = jnp.dot(oh, tri_e, preferred_element_type=jnp.float32)
    oh = oh * (ecum == 1.0).astype(jnp.float32)

    cnt = jnp.sum(oh, axis=0, keepdims=True)
    cnt_i = cnt.astype(jnp.int32)
    cnt8 = ((cnt_i + 7) & ~7).astype(jnp.float32)
    stri_e = (lax.broadcasted_iota(jnp.int32, (E, E), 0)
              < lax.broadcasted_iota(jnp.int32, (E, E), 1)).astype(jnp.float32)
    offs = jnp.dot(cnt8, stri_e, preferred_element_type=jnp.float32)

    B = 256
    tri_b = (lax.broadcasted_iota(jnp.int32, (B, B), 1)
             <= lax.broadcasted_iota(jnp.int32, (B, B), 0)).astype(jnp.float32)

    carry = jnp.zeros((1, E), jnp.float32)
    for b in range(T // B):
        r0 = b * B
        ohb = oh[r0:r0 + B, :]
        cumb = jnp.dot(tri_b, ohb, preferred_element_type=jnp.float32) + carry
        destb = jnp.sum(ohb * (offs + cumb), axis=1, keepdims=True) - 1.0
        dest_ref[r0:r0 + B, :] = destb.astype(jnp.int32)
        carry = carry + jnp.sum(ohb, axis=0, keepdims=True)

    row = jnp.concatenate([offs.astype(jnp.int32), cnt_i], axis=1)
    meta_ref[...] = jnp.broadcast_to(row, (8, 128))


def _router(router_logits):
    return pl.pallas_call(
        _router_body,
        out_shape=(
            jax.ShapeDtypeStruct((T, 1), jnp.int32),
            jax.ShapeDtypeStruct((T, 1), jnp.float32),
            jax.ShapeDtypeStruct((8, 128), jnp.int32),
        ),
    )(router_logits)


@functools.lru_cache(maxsize=None)
def _sc_kernels():
    mesh = plsc.VectorSubcoreMesh(
        core_axis_name="core", subcore_axis_name="subcore")
    bpw = T // NW

    @functools.partial(
        pl.kernel,
        out_type=jax.ShapeDtypeStruct((P, H), jnp.float32),
        mesh=mesh,
        scratch_types=[pltpu.VMEM((bpw,), jnp.int32),
                       pltpu.VMEM((bpw, H), jnp.float32)],
    )
    def dispatch(x_hbm, i_hbm, o_hbm, idx_v, rows_v):
        wid = lax.axis_index("subcore") * NC + lax.axis_index("core")
        base = wid * bpw
        pltpu.sync_copy(i_hbm.at[pl.ds(base, bpw)], idx_v)
        pltpu.sync_copy(x_hbm.at[pl.ds(base, bpw)], rows_v)
        pltpu.sync_copy(rows_v, o_hbm.at[idx_v])

    @functools.partial(
        pl.kernel,
        out_type=jax.ShapeDtypeStruct((T, H), jnp.float32),
        mesh=mesh,
        scratch_types=[pltpu.VMEM((bpw,), jnp.int32),
                       pltpu.VMEM((bpw, H), jnp.float32)],
    )
    def combine(y_hbm, i_hbm, o_hbm, idx_v, rows_v):
        wid = lax.axis_index("subcore") * NC + lax.axis_index("core")
        base = wid * bpw
        pltpu.sync_copy(i_hbm.at[pl.ds(base, bpw)], idx_v)
        pltpu.sync_copy(y_hbm.at[idx_v], rows_v)
        pltpu.sync_copy(rows_v, o_hbm.at[pl.ds(base, bpw)])

    return dispatch, combine


def _ffn_body(meta_ref, x_ref, w1_ref, w3_ref, w2_ref, y_ref):
    e = pl.program_id(0)
    i = pl.program_id(1)

    @pl.when((e == 0) & (i == 0))
    def _():
        y_ref[...] = jnp.zeros_like(y_ref)

    off = meta_ref[e]
    cnt = meta_ref[E + e]
    w1 = w1_ref[0].astype(jnp.bfloat16)
    w3 = w3_ref[0].astype(jnp.bfloat16)
    w2 = w2_ref[0].astype(jnp.bfloat16)
    nch = (cnt + CHUNK - 1) // CHUNK

    def body(c, carry):
        start = pl.multiple_of(off + c * CHUNK, 8)
        xg = x_ref[pl.ds(start, CHUNK), :].astype(jnp.bfloat16)
        a = jnp.dot(xg, w1, preferred_element_type=jnp.float32)
        b = jnp.dot(xg, w3, preferred_element_type=jnp.float32)
        h = a * (1.0 / (1.0 + jnp.exp(-a))) * b
        rid = lax.broadcasted_iota(jnp.int32, (CHUNK, BI), 0) + c * CHUNK
        h = jnp.where(rid < cnt, h, 0.0).astype(jnp.bfloat16)
        y_ref[pl.ds(start, CHUNK), :] += jnp.dot(
            h, w2, preferred_element_type=jnp.float32)
        return carry

    lax.fori_loop(0, nch, body, 0)


def _ffn(meta_vec, x_sorted, w1, w3, w2):
    ki = I // BI
    return pl.pallas_call(
        _ffn_body,
        grid=(E, ki),
        in_specs=[
            pl.BlockSpec(memory_space=pltpu.SMEM),
            pl.BlockSpec((P, H), lambda e, i: (0, 0)),
            pl.BlockSpec((1, H, BI), lambda e, i: (e, 0, i)),
            pl.BlockSpec((1, H, BI), lambda e, i: (e, 0, i)),
            pl.BlockSpec((1, BI, H), lambda e, i: (e, i, 0)),
        ],
        out_specs=pl.BlockSpec((P, H), lambda e, i: (0, 0)),
        out_shape=jax.ShapeDtypeStruct((P, H), jnp.float32),
        compiler_params=pltpu.CompilerParams(
            dimension_semantics=("arbitrary", "arbitrary")),
    )(meta_vec, x_sorted, w1, w3, w2)


def _scale_body(z_ref, g_ref, o_ref):
    o_ref[...] = z_ref[...] * g_ref[...]


def _scale(z, gate):
    return pl.pallas_call(
        _scale_body,
        grid=(8,),
        in_specs=[
            pl.BlockSpec((T // 8, H), lambda b: (b, 0)),
            pl.BlockSpec((T // 8, 1), lambda b: (b, 0)),
        ],
        out_specs=pl.BlockSpec((T // 8, H), lambda b: (b, 0)),
        out_shape=jax.ShapeDtypeStruct((T, H), jnp.float32),
    )(z, gate)


def kernel(hidden_states, router_logits, w1, w2, w3):
    dispatch, combine = _sc_kernels()
    dest, gate, meta = _router(router_logits)
    dest_row = dest.reshape(T)
    meta_vec = meta[0]
    x_sorted = dispatch(hidden_states, dest_row)
    y_sorted = _ffn(meta_vec, x_sorted, w1, w3, w2)
    z = combine(y_sorted, dest_row)
    return _scale(z, gate)


# speedup vs baseline: 1.1966x; 1.1966x over previous
import functools

import jax
import jax.numpy as jnp
from jax import lax
from jax.experimental import pallas as pl
from jax.experimental.pallas import tpu as pltpu
from jax.experimental.pallas import tpu_sc as plsc

T = 2048
E = 64
H = 768
I = 1024
P = 2560
CHUNK = 64
BI = 512
W = 64
NC = 2
NSUB = 16
NW = NC * NSUB


def _router_body(l_ref, dest_ref, gate_ref, meta_ref):
    l = l_ref[...]
    m = jnp.max(l, axis=1, keepdims=True)
    s = jnp.sum(jnp.exp(l - m), axis=1, keepdims=True)
    gate_ref[...] = 1.0 / s

    oh = (l == m).astype(jnp.float32)
    tri_e = (lax.broadcasted_iota(jnp.int32, (E, E), 0)
             <= lax.broadcasted_iota(jnp.int32, (E, E), 1)).astype(jnp.float32)
    ecum = jnp.dot(oh, tri_e, preferred_element_type=jnp.float32)
    oh = oh * (ecum == 1.0).astype(jnp.float32)

    cnt = jnp.sum(oh, axis=0, keepdims=True)
    cnt_i = cnt.astype(jnp.int32)
    cnt8 = ((cnt_i + 7) & ~7).astype(jnp.float32)
    stri_e = (lax.broadcasted_iota(jnp.int32, (E, E), 0)
              < lax.broadcasted_iota(jnp.int32, (E, E), 1)).astype(jnp.float32)
    offs = jnp.dot(cnt8, stri_e, preferred_element_type=jnp.float32)

    B = 256
    tri_b = (lax.broadcasted_iota(jnp.int32, (B, B), 1)
             <= lax.broadcasted_iota(jnp.int32, (B, B), 0)).astype(jnp.float32)

    carry = jnp.zeros((1, E), jnp.float32)
    for b in range(T // B):
        r0 = b * B
        ohb = oh[r0:r0 + B, :]
        cumb = jnp.dot(tri_b, ohb, preferred_element_type=jnp.float32) + carry
        destb = jnp.sum(ohb * (offs + cumb), axis=1, keepdims=True) - 1.0
        dest_ref[r0:r0 + B, :] = destb.astype(jnp.int32)
        carry = carry + jnp.sum(ohb, axis=0, keepdims=True)

    row = jnp.concatenate([offs.astype(jnp.int32), cnt_i], axis=1)
    meta_ref[...] = jnp.broadcast_to(row, (8, 128))


def _router(router_logits):
    return pl.pallas_call(
        _router_body,
        out_shape=(
            jax.ShapeDtypeStruct((T, 1), jnp.int32),
            jax.ShapeDtypeStruct((T, 1), jnp.float32),
            jax.ShapeDtypeStruct((8, 128), jnp.int32),
        ),
    )(router_logits)


@functools.lru_cache(maxsize=None)
def _sc_kernels():
    mesh = plsc.VectorSubcoreMesh(
        core_axis_name="core", subcore_axis_name="subcore")
    bpw = T // NW

    @functools.partial(
        pl.kernel,
        out_type=jax.ShapeDtypeStruct((P, H), jnp.float32),
        mesh=mesh,
        scratch_types=[pltpu.VMEM((bpw,), jnp.int32),
                       pltpu.VMEM((bpw, H), jnp.float32)],
    )
    def dispatch(x_hbm, i_hbm, o_hbm, idx_v, rows_v):
        wid = lax.axis_index("subcore") * NC + lax.axis_index("core")
        base = wid * bpw
        pltpu.sync_copy(i_hbm.at[pl.ds(base, bpw)], idx_v)
        pltpu.sync_copy(x_hbm.at[pl.ds(base, bpw)], rows_v)
        pltpu.sync_copy(rows_v, o_hbm.at[idx_v])

    @functools.partial(
        pl.kernel,
        out_type=jax.ShapeDtypeStruct((T, H), jnp.float32),
        mesh=mesh,
        scratch_types=[pltpu.VMEM((bpw,), jnp.int32),
                       pltpu.VMEM((bpw, H), jnp.float32)],
    )
    def combine(y_hbm, i_hbm, o_hbm, idx_v, rows_v):
        wid = lax.axis_index("subcore") * NC + lax.axis_index("core")
        base = wid * bpw
        pltpu.sync_copy(i_hbm.at[pl.ds(base, bpw)], idx_v)
        pltpu.sync_copy(y_hbm.at[idx_v], rows_v)
        pltpu.sync_copy(rows_v, o_hbm.at[pl.ds(base, bpw)])

    return dispatch, combine


def _ffn_body(meta_ref, x_ref, w1_ref, w3_ref, w2_ref, y_ref):
    e = pl.program_id(0)
    off = meta_ref[e]
    cnt = meta_ref[E + e]
    w1 = w1_ref[0].astype(jnp.bfloat16)
    w3 = w3_ref[0].astype(jnp.bfloat16)
    w2 = w2_ref[0].astype(jnp.bfloat16)
    nch = (cnt + CHUNK - 1) // CHUNK

    def body(c, carry):
        start = pl.multiple_of(off + c * CHUNK, 8)
        xg = x_ref[pl.ds(start, CHUNK), :].astype(jnp.bfloat16)
        a = jnp.dot(xg, w1, preferred_element_type=jnp.float32)
        b = jnp.dot(xg, w3, preferred_element_type=jnp.float32)
        h = a * (1.0 / (1.0 + jnp.exp(-a))) * b
        rid = lax.broadcasted_iota(jnp.int32, (CHUNK, I), 0) + c * CHUNK
        h = jnp.where(rid < cnt, h, 0.0).astype(jnp.bfloat16)
        y_ref[pl.ds(start, CHUNK), :] = jnp.dot(
            h, w2, preferred_element_type=jnp.float32)
        return carry

    lax.fori_loop(0, nch, body, 0)


def _ffn(meta_vec, x_sorted, w1, w3, w2):
    return pl.pallas_call(
        _ffn_body,
        grid=(E,),
        in_specs=[
            pl.BlockSpec(memory_space=pltpu.SMEM),
            pl.BlockSpec((P, H), lambda e: (0, 0)),
            pl.BlockSpec((1, H, I), lambda e: (e, 0, 0)),
            pl.BlockSpec((1, H, I), lambda e: (e, 0, 0)),
            pl.BlockSpec((1, I, H), lambda e: (e, 0, 0)),
        ],
        out_specs=pl.BlockSpec((P, H), lambda e: (0, 0)),
        out_shape=jax.ShapeDtypeStruct((P, H), jnp.float32),
        compiler_params=pltpu.CompilerParams(
            dimension_semantics=("arbitrary",)),
    )(meta_vec, x_sorted, w1, w3, w2)


def _scale_body(z_ref, g_ref, o_ref):
    o_ref[...] = z_ref[...] * g_ref[...]


def _scale(z, gate):
    return pl.pallas_call(
        _scale_body,
        grid=(8,),
        in_specs=[
            pl.BlockSpec((T // 8, H), lambda b: (b, 0)),
            pl.BlockSpec((T // 8, 1), lambda b: (b, 0)),
        ],
        out_specs=pl.BlockSpec((T // 8, H), lambda b: (b, 0)),
        out_shape=jax.ShapeDtypeStruct((T, H), jnp.float32),
    )(z, gate)


def kernel(hidden_states, router_logits, w1, w2, w3):
    dispatch, combine = _sc_kernels()
    dest, gate, meta = _router(router_logits)
    dest_row = dest.reshape(T)
    meta_vec = meta[0]
    x_sorted = dispatch(hidden_states, dest_row)
    y_sorted = _ffn(meta_vec, x_sorted, w1, w3, w2)
    z = combine(y_sorted, dest_row)
    return _scale(z, gate)


# speedup vs baseline: 1.2263x; 1.0248x over previous
import functools

import jax
import jax.numpy as jnp
from jax import lax
from jax.experimental import pallas as pl
from jax.experimental.pallas import tpu as pltpu
from jax.experimental.pallas import tpu_sc as plsc

T = 2048
E = 64
H = 768
I = 1024
P = 2560
CHUNK = 64
BI = 512
W = 64
NC = 2
NSUB = 16
NW = NC * NSUB


def _router_body(l_ref, dest_ref, gate_ref, meta_ref):
    l = l_ref[...]
    m = jnp.max(l, axis=1, keepdims=True)
    s = jnp.sum(jnp.exp(l - m), axis=1, keepdims=True)
    gate_ref[...] = jnp.broadcast_to(1.0 / s, (T, 128))

    oh = (l == m).astype(jnp.float32)
    tri_e = (lax.broadcasted_iota(jnp.int32, (E, E), 0)
             <= lax.broadcasted_iota(jnp.int32, (E, E), 1)).astype(jnp.float32)
    ecum = jnp.dot(oh, tri_e, preferred_element_type=jnp.float32)
    oh = oh * (ecum == 1.0).astype(jnp.float32)

    cnt = jnp.sum(oh, axis=0, keepdims=True)
    cnt_i = cnt.astype(jnp.int32)
    cnt8 = ((cnt_i + 7) & ~7).astype(jnp.float32)
    stri_e = (lax.broadcasted_iota(jnp.int32, (E, E), 0)
              < lax.broadcasted_iota(jnp.int32, (E, E), 1)).astype(jnp.float32)
    offs = jnp.dot(cnt8, stri_e, preferred_element_type=jnp.float32)

    B = 256
    tri_b = (lax.broadcasted_iota(jnp.int32, (B, B), 1)
             <= lax.broadcasted_iota(jnp.int32, (B, B), 0)).astype(jnp.float32)

    carry = jnp.zeros((1, E), jnp.float32)
    for b in range(T // B):
        r0 = b * B
        ohb = oh[r0:r0 + B, :]
        cumb = jnp.dot(tri_b, ohb, preferred_element_type=jnp.float32) + carry
        destb = jnp.sum(ohb * (offs + cumb), axis=1, keepdims=True) - 1.0
        dest_ref[r0:r0 + B, :] = destb.astype(jnp.int32)
        carry = carry + jnp.sum(ohb, axis=0, keepdims=True)

    row = jnp.concatenate([offs.astype(jnp.int32), cnt_i], axis=1)
    meta_ref[...] = jnp.broadcast_to(row, (8, 128))


def _router(router_logits):
    return pl.pallas_call(
        _router_body,
        out_shape=(
            jax.ShapeDtypeStruct((T, 1), jnp.int32),
            jax.ShapeDtypeStruct((T, 128), jnp.float32),
            jax.ShapeDtypeStruct((8, 128), jnp.int32),
        ),
    )(router_logits)


@functools.lru_cache(maxsize=None)
def _sc_kernels():
    mesh = plsc.VectorSubcoreMesh(
        core_axis_name="core", subcore_axis_name="subcore")
    bpw = T // NW

    @functools.partial(
        pl.kernel,
        out_type=(jax.ShapeDtypeStruct((P, H), jnp.float32),
                  jax.ShapeDtypeStruct((P, 128), jnp.float32)),
        mesh=mesh,
        scratch_types=[pltpu.VMEM((bpw,), jnp.int32),
                       pltpu.VMEM((bpw, H), jnp.float32),
                       pltpu.VMEM((bpw, 128), jnp.float32)],
    )
    def dispatch(x_hbm, g_hbm, i_hbm, o_hbm, og_hbm, idx_v, rows_v, g_v):
        wid = lax.axis_index("subcore") * NC + lax.axis_index("core")
        base = wid * bpw
        pltpu.sync_copy(i_hbm.at[pl.ds(base, bpw)], idx_v)
        pltpu.sync_copy(x_hbm.at[pl.ds(base, bpw)], rows_v)
        pltpu.sync_copy(g_hbm.at[pl.ds(base, bpw)], g_v)
        pltpu.sync_copy(rows_v, o_hbm.at[idx_v])
        pltpu.sync_copy(g_v, og_hbm.at[idx_v])

    @functools.partial(
        pl.kernel,
        out_type=jax.ShapeDtypeStruct((T, H), jnp.float32),
        mesh=mesh,
        scratch_types=[pltpu.VMEM((bpw,), jnp.int32),
                       pltpu.VMEM((bpw, H), jnp.float32)],
    )
    def combine(y_hbm, i_hbm, o_hbm, idx_v, rows_v):
        wid = lax.axis_index("subcore") * NC + lax.axis_index("core")
        base = wid * bpw
        pltpu.sync_copy(i_hbm.at[pl.ds(base, bpw)], idx_v)
        pltpu.sync_copy(y_hbm.at[idx_v], rows_v)
        pltpu.sync_copy(rows_v, o_hbm.at[pl.ds(base, bpw)])

    return dispatch, combine


def _ffn_body(meta_ref, x_ref, g_ref, w1_ref, w3_ref, w2_ref, y_ref):
    e = pl.program_id(0)
    off = meta_ref[e]
    cnt = meta_ref[E + e]
    w1 = w1_ref[0].astype(jnp.bfloat16)
    w3 = w3_ref[0].astype(jnp.bfloat16)
    w2 = w2_ref[0].astype(jnp.bfloat16)
    nch = (cnt + CHUNK - 1) // CHUNK

    def body(c, carry):
        start = pl.multiple_of(off + c * CHUNK, 8)
        xg = x_ref[pl.ds(start, CHUNK), :].astype(jnp.bfloat16)
        a = jnp.dot(xg, w1, preferred_element_type=jnp.float32)
        b = jnp.dot(xg, w3, preferred_element_type=jnp.float32)
        g = g_ref[pl.ds(start, CHUNK), 0:1]
        h = a * (1.0 / (1.0 + jnp.exp(-a))) * b * g
        rid = lax.broadcasted_iota(jnp.int32, (CHUNK, I), 0) + c * CHUNK
        h = jnp.where(rid < cnt, h, 0.0).astype(jnp.bfloat16)
        y_ref[pl.ds(start, CHUNK), :] = jnp.dot(
            h, w2, preferred_element_type=jnp.float32)
        return carry

    lax.fori_loop(0, nch, body, 0)


def _ffn(meta_vec, x_sorted, gate_sorted, w1, w3, w2):
    return pl.pallas_call(
        _ffn_body,
        grid=(E,),
        in_specs=[
            pl.BlockSpec(memory_space=pltpu.SMEM),
            pl.BlockSpec((P, H), lambda e: (0, 0)),
            pl.BlockSpec((P, 128), lambda e: (0, 0)),
            pl.BlockSpec((1, H, I), lambda e: (e, 0, 0)),
            pl.BlockSpec((1, H, I), lambda e: (e, 0, 0)),
            pl.BlockSpec((1, I, H), lambda e: (e, 0, 0)),
        ],
        out_specs=pl.BlockSpec((P, H), lambda e: (0, 0)),
        out_shape=jax.ShapeDtypeStruct((P, H), jnp.float32),
        compiler_params=pltpu.CompilerParams(
            dimension_semantics=("arbitrary",)),
    )(meta_vec, x_sorted, gate_sorted, w1, w3, w2)


def kernel(hidden_states, router_logits, w1, w2, w3):
    dispatch, combine = _sc_kernels()
    dest, gate, meta = _router(router_logits)
    dest_row = dest.reshape(T)
    meta_vec = meta[0]
    x_sorted, gate_sorted = dispatch(hidden_states, gate, dest_row)
    y_sorted = _ffn(meta_vec, x_sorted, gate_sorted, w1, w3, w2)
    return combine(y_sorted, dest_row)
